# Initial kernel scaffold; baseline (speedup 1.0000x reference)
import jax
import jax.numpy as jnp
from jax.experimental import pallas as pl


def kernel(inputs, targets, loss_weights):
    raise NotImplementedError("write your pallas kernel here")


# speedup vs baseline: 1.1025x; 1.1025x over previous
import functools

import jax
import jax.numpy as jnp
from jax import lax
from jax.experimental import pallas as pl
from jax.experimental.pallas import tpu as pltpu
from jax.experimental.pallas import tpu_sc as plsc

_B = 16384
_D = 1000


_G = 16
_TB = _B // _G


def _dense_body(w_ref, x_ref, out_ref):
    i = pl.program_id(0)

    @pl.when(i == 0)
    def _init():
        out_ref[...] = jnp.zeros((1, 1), jnp.float32)

    x = x_ref[...]
    out_ref[...] = out_ref[...] + jnp.sum(w_ref[...] * x * x)


def _dense_sum(x, w):
    return pl.pallas_call(
        _dense_body,
        grid=(_G,),
        in_specs=[
            pl.BlockSpec((1, _D), lambda i: (0, 0)),
            pl.BlockSpec((_TB, _D), lambda i: (i, 0)),
        ],
        out_specs=pl.BlockSpec((1, 1), lambda i: (0, 0)),
        out_shape=jax.ShapeDtypeStruct((1, 1), jnp.float32),
    )(w.reshape(1, _D), x)


_NC = 2
_NS = 16
_NW = _NC * _NS
_BPW = _B // _NW
_NCHUNK = _BPW // 16
_NIDX = _BPW // 128


@functools.partial(
    pl.kernel,
    mesh=plsc.VectorSubcoreMesh(core_axis_name="c", subcore_axis_name="s"),
    out_type=jax.ShapeDtypeStruct((_NW, 16), jnp.float32),
    scratch_types=[
        pltpu.VMEM((_BPW,), jnp.int32),
        pltpu.VMEM((_NIDX, 128), jnp.int32),
        pltpu.VMEM((_NIDX, 128), jnp.int32),
        pltpu.VMEM((_NIDX, 128), jnp.float32),
        pltpu.VMEM((_NIDX, 128), jnp.float32),
        pltpu.VMEM((16,), jnp.float32),
        pltpu.SemaphoreType.DMA,
    ],
)
def _corr_kernel(xflat_hbm, tgt_hbm, w_hbm, out_hbm,
                 tgt_v, idx_v, tdx_v, xs_v, ws_v, o_v, sem):
    wid = lax.axis_index("s") * _NC + lax.axis_index("c")
    base = wid * _BPW
    pltpu.sync_copy(tgt_hbm.at[pl.ds(base, _BPW)], tgt_v)

    for i in range(_NCHUNK):
        t16 = tgt_v[pl.ds(i * 16, 16)]
        rows = base + i * 16 + lax.broadcasted_iota(jnp.int32, (16,), 0)
        idx_v[i // 8, pl.ds((i % 8) * 16, 16)] = rows * _D + t16
        tdx_v[i // 8, pl.ds((i % 8) * 16, 16)] = t16

    copies = [
        pltpu.async_copy(xflat_hbm.at[idx_v.at[j]], xs_v.at[j], sem)
        for j in range(_NIDX)
    ] + [
        pltpu.async_copy(w_hbm.at[tdx_v.at[j]], ws_v.at[j], sem)
        for j in range(_NIDX)
    ]
    for cp in copies:
        cp.wait()

    acc = jnp.zeros((16,), jnp.float32)
    for i in range(_NCHUNK):
        x16 = xs_v[i // 8, pl.ds((i % 8) * 16, 16)]
        w16 = ws_v[i // 8, pl.ds((i % 8) * 16, 16)]
        acc = acc + w16 * (1.0 - 2.0 * x16)
    o_v[...] = acc
    pltpu.sync_copy(o_v, out_hbm.at[wid])


def kernel(inputs, targets, loss_weights):
    dense = _dense_sum(inputs, loss_weights)
    corr = _corr_kernel(inputs.reshape(_B * _D), targets, loss_weights)
    total = dense[0, 0] + jnp.sum(corr)
    return total / jnp.float32(_B * _D)
